# baseline (speedup 1.0000x reference)
import functools
import math

import jax
import jax.numpy as jnp
from jax.experimental import pallas as pl
from jax.experimental.pallas import tpu as pltpu

EPS = 1e-8
B, S, D = 2, 2048, 1024
NH = 4
DH = D // NH
BLK = 4
NQKV = D // BLK
T = 256
NBLK = S // T
LN_EPS = 1e-5
INV_SQRT_DH = 1.0 / math.sqrt(DH)


def _scan_lanes(v, op, fill):
    r, s = v.shape
    d = 1
    while d < s:
        pad = jnp.full((r, d), fill, dtype=v.dtype)
        v = op(v, jnp.concatenate([pad, v[:, : s - d]], axis=1))
        d *= 2
    return v


def _gate_kernel(x_ref, wg_ref, bg_ref, o_ref):
    t = jax.lax.dot_general(
        wg_ref[...], x_ref[0], (((0,), (1,)), ((), ())),
        preferred_element_type=jnp.float32)
    t = t + bg_ref[...]
    ig = t[0:NH]
    fg = t[NH:2 * NH]
    lf = jnp.minimum(fg, 0.0) - jnp.log1p(jnp.exp(-jnp.abs(fg)))
    cs = _scan_lanes(lf, jnp.add, 0.0)
    m = ig - cs
    mx = _scan_lanes(m, jnp.maximum, -1e30)
    nf = jnp.exp(-(cs + mx))
    rows = jnp.concatenate(
        [jnp.concatenate([m[h:h + 1], mx[h:h + 1], nf[h:h + 1]], axis=0)
         for h in range(NH)], axis=0)
    ii = jax.lax.broadcasted_iota(jnp.int32, (T, T), 0)
    jj = jax.lax.broadcasted_iota(jnp.int32, (T, T), 1)
    eye = jnp.where(ii == jj, 1.0, 0.0).astype(jnp.float32)
    for c in range(NBLK):
        chunk = rows[:, c * T:(c + 1) * T]
        t2 = jax.lax.dot_general(
            eye, chunk, (((1,), (1,)), ((), ())),
            preferred_element_type=jnp.float32,
            precision=jax.lax.Precision.HIGHEST)
        for h in range(NH):
            o_ref[0, h, c * T:(c + 1) * T, :] = t2[:, 3 * h:3 * h + 3]


def _mlstm_kernel(x_ref, wq_ref, wk_ref, wv_ref, g_ref, gam_ref, o_ref,
                  ke_s, v_s):
    ii = jax.lax.broadcasted_iota(jnp.int32, (T, T), 0)
    jj = jax.lax.broadcasted_iota(jnp.int32, (T, T), 1)
    causal = ii >= jj

    maxm = []
    for J in range(NBLK):
        sl = slice(J * T, (J + 1) * T)
        mJ = g_ref[0, 0, sl, 0:1]
        mx = jnp.max(mJ)
        maxm.append(mx)
        e = jnp.exp(mJ - mx)
        kJ = jnp.dot(x_ref[0, sl, :], wk_ref[...],
                     preferred_element_type=jnp.float32)
        ke_s[sl, :] = kJ * e
        v_s[sl, :] = jnp.dot(x_ref[0, sl, :], wv_ref[...],
                             preferred_element_type=jnp.float32)

    for I in range(NBLK):
        sli = slice(I * T, (I + 1) * T)
        qI = jnp.dot(x_ref[0, sli, :], wq_ref[...],
                     preferred_element_type=jnp.float32) * INV_SQRT_DH
        MI = g_ref[0, 0, sli, 1:2]
        nfI = g_ref[0, 0, sli, 2:3]
        acc = jnp.zeros((T, DH), jnp.float32)
        ssum = jnp.zeros((T, 1), jnp.float32)
        for J in range(I + 1):
            slj = slice(J * T, (J + 1) * T)
            s = jax.lax.dot_general(
                qI, ke_s[slj, :], (((1,), (1,)), ((), ())),
                preferred_element_type=jnp.float32)
            rf = jnp.exp(jnp.minimum(maxm[J] - MI, 80.0))
            c = s * rf
            if J == I:
                c = jnp.where(causal, c, 0.0)
            ssum = ssum + jnp.sum(c, axis=1, keepdims=True)
            acc = acc + jnp.dot(c, v_s[slj, :],
                                preferred_element_type=jnp.float32)
        norm = jnp.maximum(jnp.abs(ssum), nfI) + EPS
        hI = acc / norm
        mu = jnp.mean(hI, axis=1, keepdims=True)
        var = jnp.mean((hI - mu) * (hI - mu), axis=1, keepdims=True)
        hn = (hI - mu) * jax.lax.rsqrt(var + LN_EPS)
        o_ref[0, sli, :] = hn * gam_ref[0]


@jax.jit
def kernel(x, wq, wk, wv, wi, bi, wf, bf, ln_w):
    f32 = jnp.float32
    x = x.astype(f32)
    eye = jnp.eye(NQKV, dtype=f32)
    wq_d = jnp.einsum('hoi,hg->higo', wq.astype(f32), eye).reshape(D, D)
    wk_d = jnp.einsum('hoi,hg->higo', wk.astype(f32), eye).reshape(D, D)
    wv_d = jnp.einsum('hoi,hg->higo', wv.astype(f32), eye).reshape(D, D)
    wi_eff = wi[:, :D] + wi[:, D:2 * D] + wi[:, 2 * D:]
    wf_eff = wf[:, :D] + wf[:, D:2 * D] + wf[:, 2 * D:]
    wg = jnp.concatenate([wi_eff, wf_eff], axis=0).T.astype(f32)
    bg = jnp.concatenate([bi, bf]).reshape(2 * NH, 1).astype(f32)
    gam = (1.0 + ln_w).reshape(NH, 1, DH).astype(f32)

    gstats = pl.pallas_call(
        _gate_kernel,
        out_shape=jax.ShapeDtypeStruct((B, NH, S, 3), f32),
        grid=(B,),
        in_specs=[
            pl.BlockSpec((1, S, D), lambda i: (i, 0, 0)),
            pl.BlockSpec((D, 2 * NH), lambda i: (0, 0)),
            pl.BlockSpec((2 * NH, 1), lambda i: (0, 0)),
        ],
        out_specs=pl.BlockSpec((1, NH, S, 3), lambda i: (i, 0, 0, 0)),
        compiler_params=pltpu.CompilerParams(
            dimension_semantics=("parallel",),
            vmem_limit_bytes=56 * 1024 * 1024,
        ),
        name="mlstm_gates",
    )(x, wg, bg)

    out = pl.pallas_call(
        _mlstm_kernel,
        out_shape=jax.ShapeDtypeStruct((B, S, D), f32),
        grid=(B * NH,),
        in_specs=[
            pl.BlockSpec((1, S, D), lambda i: (i // NH, 0, 0)),
            pl.BlockSpec((D, DH), lambda i: (0, i % NH)),
            pl.BlockSpec((D, DH), lambda i: (0, i % NH)),
            pl.BlockSpec((D, DH), lambda i: (0, i % NH)),
            pl.BlockSpec((1, 1, S, 3), lambda i: (i // NH, i % NH, 0, 0)),
            pl.BlockSpec((1, 1, DH), lambda i: (i % NH, 0, 0)),
        ],
        out_specs=pl.BlockSpec((1, S, DH), lambda i: (i // NH, 0, i % NH)),
        scratch_shapes=[
            pltpu.VMEM((S, DH), f32),
            pltpu.VMEM((S, DH), f32),
        ],
        compiler_params=pltpu.CompilerParams(
            dimension_semantics=("parallel",),
            vmem_limit_bytes=56 * 1024 * 1024,
        ),
        name="mlstm_attn",
    )(x, wq_d, wk_d, wv_d, gstats, gam)
    return out


# speedup vs baseline: 2.4567x; 2.4567x over previous
import functools
import math

import jax
import jax.numpy as jnp
from jax.experimental import pallas as pl
from jax.experimental.pallas import tpu as pltpu

EPS = 1e-8
B, S, D = 2, 2048, 1024
NH = 4
DH = D // NH
BLK = 4
NQKV = D // BLK
T = 256
NBLK = S // T
LN_EPS = 1e-5
INV_SQRT_DH = 1.0 / math.sqrt(DH)


def _scan_lanes(v, op, fill):
    r, s = v.shape
    d = 1
    while d < s:
        pad = jnp.full((r, d), fill, dtype=v.dtype)
        v = op(v, jnp.concatenate([pad, v[:, : s - d]], axis=1))
        d *= 2
    return v


def _gate_kernel(x_ref, wg_ref, bg_ref, o_ref):
    t = jax.lax.dot_general(
        wg_ref[...], x_ref[0], (((0,), (1,)), ((), ())),
        preferred_element_type=jnp.float32)
    t = t + bg_ref[...]
    ig = t[0:NH]
    fg = t[NH:2 * NH]
    lf = jnp.minimum(fg, 0.0) - jnp.log1p(jnp.exp(-jnp.abs(fg)))
    cs = _scan_lanes(lf, jnp.add, 0.0)
    m = ig - cs
    mx = _scan_lanes(m, jnp.maximum, -1e30)
    nf = jnp.exp(-(cs + mx))
    rows = jnp.concatenate(
        [jnp.concatenate([m[h:h + 1], mx[h:h + 1], nf[h:h + 1]], axis=0)
         for h in range(NH)], axis=0)
    ii = jax.lax.broadcasted_iota(jnp.int32, (T, T), 0)
    jj = jax.lax.broadcasted_iota(jnp.int32, (T, T), 1)
    eye = jnp.where(ii == jj, 1.0, 0.0).astype(jnp.float32)
    for c in range(NBLK):
        chunk = rows[:, c * T:(c + 1) * T]
        t2 = jax.lax.dot_general(
            eye, chunk, (((1,), (1,)), ((), ())),
            preferred_element_type=jnp.float32,
            precision=jax.lax.Precision.HIGHEST)
        for h in range(NH):
            o_ref[0, h, c * T:(c + 1) * T, :] = t2[:, 3 * h:3 * h + 3]


def _mlstm_kernel(x_ref, wq_ref, wk_ref, wv_ref, g_ref, gam_ref, o_ref,
                  ke_s, v_s):
    ii = jax.lax.broadcasted_iota(jnp.int32, (T, T), 0)
    jj = jax.lax.broadcasted_iota(jnp.int32, (T, T), 1)
    causal = ii >= jj

    maxm = []
    for J in range(NBLK):
        sl = slice(J * T, (J + 1) * T)
        mJ = g_ref[0, 0, sl, 0:1]
        mx = jnp.max(mJ)
        maxm.append(mx)
        e = jnp.exp(mJ - mx)
        kJ = jnp.dot(x_ref[0, sl, :], wk_ref[0],
                     preferred_element_type=jnp.float32)
        ke_s[sl, :] = kJ * e
        v_s[sl, :] = jnp.dot(x_ref[0, sl, :], wv_ref[0],
                             preferred_element_type=jnp.float32)

    for I in range(NBLK):
        sli = slice(I * T, (I + 1) * T)
        qI = jnp.dot(x_ref[0, sli, :], wq_ref[0],
                     preferred_element_type=jnp.float32) * INV_SQRT_DH
        MI = g_ref[0, 0, sli, 1:2]
        nfI = g_ref[0, 0, sli, 2:3]
        acc = jnp.zeros((T, DH), jnp.float32)
        ssum = jnp.zeros((T, 1), jnp.float32)
        for J in range(I + 1):
            slj = slice(J * T, (J + 1) * T)
            s = jax.lax.dot_general(
                qI, ke_s[slj, :], (((1,), (1,)), ((), ())),
                preferred_element_type=jnp.float32)
            rf = jnp.exp(jnp.minimum(maxm[J] - MI, 80.0))
            c = s * rf
            if J == I:
                c = jnp.where(causal, c, 0.0)
            ssum = ssum + jnp.sum(c, axis=1, keepdims=True)
            acc = acc + jnp.dot(c, v_s[slj, :],
                                preferred_element_type=jnp.float32)
        norm = jnp.maximum(jnp.abs(ssum), nfI) + EPS
        hI = acc / norm
        mu = jnp.mean(hI, axis=1, keepdims=True)
        var = jnp.mean((hI - mu) * (hI - mu), axis=1, keepdims=True)
        hn = (hI - mu) * jax.lax.rsqrt(var + LN_EPS)
        o_ref[0, sli, :] = hn * gam_ref[0]


@jax.jit
def kernel(x, wq, wk, wv, wi, bi, wf, bf, ln_w):
    f32 = jnp.float32
    x = x.astype(f32)
    nhb = DH // BLK
    eye = jnp.eye(nhb, dtype=f32)[None, :, None, :, None]

    def _blockdiag(w):
        wt = w.astype(f32).reshape(NH, nhb, BLK, BLK).transpose(0, 1, 3, 2)
        return (wt[:, :, :, None, :] * eye).reshape(NH, DH, DH)

    wq_d = _blockdiag(wq)
    wk_d = _blockdiag(wk)
    wv_d = _blockdiag(wv)
    wi_eff = wi[:, :D] + wi[:, D:2 * D] + wi[:, 2 * D:]
    wf_eff = wf[:, :D] + wf[:, D:2 * D] + wf[:, 2 * D:]
    wg = jnp.concatenate([wi_eff, wf_eff], axis=0).T.astype(f32)
    bg = jnp.concatenate([bi, bf]).reshape(2 * NH, 1).astype(f32)
    gam = (1.0 + ln_w).reshape(NH, 1, DH).astype(f32)

    gstats = pl.pallas_call(
        _gate_kernel,
        out_shape=jax.ShapeDtypeStruct((B, NH, S, 3), f32),
        grid=(B,),
        in_specs=[
            pl.BlockSpec((1, S, D), lambda i: (i, 0, 0)),
            pl.BlockSpec((D, 2 * NH), lambda i: (0, 0)),
            pl.BlockSpec((2 * NH, 1), lambda i: (0, 0)),
        ],
        out_specs=pl.BlockSpec((1, NH, S, 3), lambda i: (i, 0, 0, 0)),
        compiler_params=pltpu.CompilerParams(
            dimension_semantics=("parallel",),
            vmem_limit_bytes=56 * 1024 * 1024,
        ),
        name="mlstm_gates",
    )(x, wg, bg)

    out = pl.pallas_call(
        _mlstm_kernel,
        out_shape=jax.ShapeDtypeStruct((B, S, D), f32),
        grid=(B * NH,),
        in_specs=[
            pl.BlockSpec((1, S, DH), lambda i: (i // NH, 0, i % NH)),
            pl.BlockSpec((1, DH, DH), lambda i: (i % NH, 0, 0)),
            pl.BlockSpec((1, DH, DH), lambda i: (i % NH, 0, 0)),
            pl.BlockSpec((1, DH, DH), lambda i: (i % NH, 0, 0)),
            pl.BlockSpec((1, 1, S, 3), lambda i: (i // NH, i % NH, 0, 0)),
            pl.BlockSpec((1, 1, DH), lambda i: (i % NH, 0, 0)),
        ],
        out_specs=pl.BlockSpec((1, S, DH), lambda i: (i // NH, 0, i % NH)),
        scratch_shapes=[
            pltpu.VMEM((S, DH), f32),
            pltpu.VMEM((S, DH), f32),
        ],
        compiler_params=pltpu.CompilerParams(
            dimension_semantics=("parallel",),
            vmem_limit_bytes=56 * 1024 * 1024,
        ),
        name="mlstm_attn",
    )(x, wq_d, wk_d, wv_d, gstats, gam)
    return out


# speedup vs baseline: 5.4115x; 2.2028x over previous
import functools
import math

import jax
import jax.numpy as jnp
from jax.experimental import pallas as pl
from jax.experimental.pallas import tpu as pltpu

EPS = 1e-8
B, S, D = 2, 2048, 1024
NH = 4
DH = D // NH
BLK = 4
NQKV = D // BLK
T = 256
NBLK = S // T
LN_EPS = 1e-5
INV_SQRT_DH = 1.0 / math.sqrt(DH)


def _scan_lanes(v, op, fill):
    r, s = v.shape
    d = 1
    while d < s:
        pad = jnp.full((r, d), fill, dtype=v.dtype)
        v = op(v, jnp.concatenate([pad, v[:, : s - d]], axis=1))
        d *= 2
    return v


def _gate_kernel(x_ref, wg_ref, bg_ref, o_ref):
    t = jax.lax.dot_general(
        wg_ref[...], x_ref[0], (((0,), (1,)), ((), ())),
        preferred_element_type=jnp.float32)
    t = t + bg_ref[...]
    ig = t[0:NH]
    fg = t[NH:2 * NH]
    lf = jnp.minimum(fg, 0.0) - jnp.log1p(jnp.exp(-jnp.abs(fg)))
    cs = _scan_lanes(lf, jnp.add, 0.0)
    m = ig - cs
    mx = _scan_lanes(m, jnp.maximum, -1e30)
    nf = jnp.exp(-(cs + mx))
    rows = jnp.concatenate(
        [jnp.concatenate([m[h:h + 1], mx[h:h + 1], nf[h:h + 1]], axis=0)
         for h in range(NH)], axis=0)
    ii = jax.lax.broadcasted_iota(jnp.int32, (T, T), 0)
    jj = jax.lax.broadcasted_iota(jnp.int32, (T, T), 1)
    eye = jnp.where(ii == jj, 1.0, 0.0).astype(jnp.float32)
    for c in range(NBLK):
        chunk = rows[:, c * T:(c + 1) * T]
        t2 = jax.lax.dot_general(
            eye, chunk, (((1,), (1,)), ((), ())),
            preferred_element_type=jnp.float32,
            precision=jax.lax.Precision.HIGHEST)
        for h in range(NH):
            o_ref[0, h, c * T:(c + 1) * T, :] = t2[:, 3 * h:3 * h + 3]


def _mlstm_kernel(x_ref, wq_ref, wk_ref, wv_ref, g_ref, gam_ref, o_ref,
                  ke_s, v_s):
    ii = jax.lax.broadcasted_iota(jnp.int32, (T, T), 0)
    jj = jax.lax.broadcasted_iota(jnp.int32, (T, T), 1)
    causal = ii >= jj

    maxm = []
    for J in range(NBLK):
        sl = slice(J * T, (J + 1) * T)
        mJ = g_ref[0, 0, sl, 0:1]
        mx = jnp.max(mJ)
        maxm.append(mx)
        e = jnp.exp(mJ - mx)
        kJ = jnp.dot(x_ref[0, sl, :], wk_ref[0],
                     preferred_element_type=jnp.float32)
        ke_s[sl, :] = kJ * e
        v_s[sl, :] = jnp.dot(x_ref[0, sl, :], wv_ref[0],
                             preferred_element_type=jnp.float32)

    for I in range(NBLK):
        sli = slice(I * T, (I + 1) * T)
        qI = jnp.dot(x_ref[0, sli, :], wq_ref[0],
                     preferred_element_type=jnp.float32) * INV_SQRT_DH
        MI = g_ref[0, 0, sli, 1:2]
        nfI = g_ref[0, 0, sli, 2:3]
        acc = jnp.zeros((T, DH), jnp.float32)
        ssum = jnp.zeros((T, 1), jnp.float32)
        for J in range(I + 1):
            slj = slice(J * T, (J + 1) * T)
            s = jax.lax.dot_general(
                qI, ke_s[slj, :], (((1,), (1,)), ((), ())),
                preferred_element_type=jnp.float32)
            rf = jnp.exp(jnp.minimum(maxm[J] - MI, 80.0))
            c = s * rf
            if J == I:
                c = jnp.where(causal, c, 0.0)
            ssum = ssum + jnp.sum(c, axis=1, keepdims=True)
            acc = acc + jnp.dot(c, v_s[slj, :],
                                preferred_element_type=jnp.float32)
        norm = jnp.maximum(jnp.abs(ssum), nfI) + EPS
        hI = acc / norm
        mu = jnp.mean(hI, axis=1, keepdims=True)
        var = jnp.mean((hI - mu) * (hI - mu), axis=1, keepdims=True)
        hn = (hI - mu) * jax.lax.rsqrt(var + LN_EPS)
        o_ref[0, sli, :] = hn * gam_ref[0]


@jax.jit
def kernel(x, wq, wk, wv, wi, bi, wf, bf, ln_w):
    f32 = jnp.float32
    x = x.astype(f32)
    nhb = DH // BLK

    def _blockdiag(w):
        wt = w.astype(f32).reshape(NH, nhb, BLK, BLK).transpose(0, 1, 3, 2)
        p1 = jnp.pad(wt, ((0, 0), (0, 0), (0, 0), (0, (nhb - 1) * BLK)))
        p2 = p1.reshape(NH, nhb, BLK * DH)
        p3 = jnp.pad(p2, ((0, 0), (0, 0), (0, BLK)))
        p4 = p3.reshape(NH, nhb * (BLK * DH + BLK))[:, :DH * DH]
        return p4.reshape(NH, DH, DH)

    wq_d = _blockdiag(wq)
    wk_d = _blockdiag(wk)
    wv_d = _blockdiag(wv)
    wi_eff = wi[:, :D] + wi[:, D:2 * D] + wi[:, 2 * D:]
    wf_eff = wf[:, :D] + wf[:, D:2 * D] + wf[:, 2 * D:]
    wg = jnp.concatenate([wi_eff, wf_eff], axis=0).T.astype(f32)
    bg = jnp.concatenate([bi, bf]).reshape(2 * NH, 1).astype(f32)
    gam = (1.0 + ln_w).reshape(NH, 1, DH).astype(f32)

    gstats = pl.pallas_call(
        _gate_kernel,
        out_shape=jax.ShapeDtypeStruct((B, NH, S, 3), f32),
        grid=(B,),
        in_specs=[
            pl.BlockSpec((1, S, D), lambda i: (i, 0, 0)),
            pl.BlockSpec((D, 2 * NH), lambda i: (0, 0)),
            pl.BlockSpec((2 * NH, 1), lambda i: (0, 0)),
        ],
        out_specs=pl.BlockSpec((1, NH, S, 3), lambda i: (i, 0, 0, 0)),
        compiler_params=pltpu.CompilerParams(
            dimension_semantics=("parallel",),
            vmem_limit_bytes=56 * 1024 * 1024,
        ),
        name="mlstm_gates",
    )(x, wg, bg)

    out = pl.pallas_call(
        _mlstm_kernel,
        out_shape=jax.ShapeDtypeStruct((B, S, D), f32),
        grid=(B * NH,),
        in_specs=[
            pl.BlockSpec((1, S, DH), lambda i: (i // NH, 0, i % NH)),
            pl.BlockSpec((1, DH, DH), lambda i: (i % NH, 0, 0)),
            pl.BlockSpec((1, DH, DH), lambda i: (i % NH, 0, 0)),
            pl.BlockSpec((1, DH, DH), lambda i: (i % NH, 0, 0)),
            pl.BlockSpec((1, 1, S, 3), lambda i: (i // NH, i % NH, 0, 0)),
            pl.BlockSpec((1, 1, DH), lambda i: (i % NH, 0, 0)),
        ],
        out_specs=pl.BlockSpec((1, S, DH), lambda i: (i // NH, 0, i % NH)),
        scratch_shapes=[
            pltpu.VMEM((S, DH), f32),
            pltpu.VMEM((S, DH), f32),
        ],
        compiler_params=pltpu.CompilerParams(
            dimension_semantics=("parallel",),
            vmem_limit_bytes=56 * 1024 * 1024,
        ),
        name="mlstm_attn",
    )(x, wq_d, wk_d, wv_d, gstats, gam)
    return out


# speedup vs baseline: 5.7166x; 1.0564x over previous
import functools
import math

import jax
import jax.numpy as jnp
from jax.experimental import pallas as pl
from jax.experimental.pallas import tpu as pltpu

EPS = 1e-8
B, S, D = 2, 2048, 1024
NH = 4
DH = D // NH
BLK = 4
NQKV = D // BLK
T = 256
NBLK = S // T
LN_EPS = 1e-5
INV_SQRT_DH = 1.0 / math.sqrt(DH)


def _scan_lanes(v, op, fill):
    r, s = v.shape
    d = 1
    while d < s:
        pad = jnp.full((r, d), fill, dtype=v.dtype)
        v = op(v, jnp.concatenate([pad, v[:, : s - d]], axis=1))
        d *= 2
    return v


def _gate_kernel(x_ref, wg_ref, bg_ref, o_ref):
    t = jax.lax.dot_general(
        wg_ref[...], x_ref[0], (((0,), (1,)), ((), ())),
        preferred_element_type=jnp.float32)
    t = t + bg_ref[...]
    ig = t[0:NH]
    fg = t[NH:2 * NH]
    lf = jnp.minimum(fg, 0.0) - jnp.log1p(jnp.exp(-jnp.abs(fg)))
    cs = _scan_lanes(lf, jnp.add, 0.0)
    m = ig - cs
    mx = _scan_lanes(m, jnp.maximum, -1e30)
    nf = jnp.exp(-(cs + mx))
    rows = jnp.concatenate(
        [jnp.concatenate([m[h:h + 1], mx[h:h + 1], nf[h:h + 1]], axis=0)
         for h in range(NH)], axis=0)
    t2 = jnp.swapaxes(rows, 0, 1)
    for h in range(NH):
        o_ref[0, h, :, :] = t2[:, 3 * h:3 * h + 3]


def _mlstm_kernel(x_ref, wq_ref, wk_ref, wv_ref, g_ref, gam_ref, o_ref,
                  ke_s, v_s):
    ii = jax.lax.broadcasted_iota(jnp.int32, (T, T), 0)
    jj = jax.lax.broadcasted_iota(jnp.int32, (T, T), 1)
    causal = ii >= jj

    maxm = []
    for J in range(NBLK):
        sl = slice(J * T, (J + 1) * T)
        mJ = g_ref[0, 0, sl, 0:1]
        mx = jnp.max(mJ)
        maxm.append(mx)
        e = jnp.exp(mJ - mx)
        xb = x_ref[0, sl, :].astype(jnp.bfloat16)
        kJ = jnp.dot(xb, wk_ref[0], preferred_element_type=jnp.float32)
        ke_s[sl, :] = (kJ * e).astype(jnp.bfloat16)
        v_s[sl, :] = jnp.dot(xb, wv_ref[0],
                             preferred_element_type=jnp.float32
                             ).astype(jnp.bfloat16)

    for I in range(NBLK):
        sli = slice(I * T, (I + 1) * T)
        qI = (jnp.dot(x_ref[0, sli, :].astype(jnp.bfloat16), wq_ref[0],
                      preferred_element_type=jnp.float32)
              * INV_SQRT_DH).astype(jnp.bfloat16)
        MI = g_ref[0, 0, sli, 1:2]
        nfI = g_ref[0, 0, sli, 2:3]
        acc = jnp.zeros((T, DH), jnp.float32)
        ssum = jnp.zeros((T, 1), jnp.float32)
        for J in range(I + 1):
            slj = slice(J * T, (J + 1) * T)
            s = jax.lax.dot_general(
                qI, ke_s[slj, :], (((1,), (1,)), ((), ())),
                preferred_element_type=jnp.float32)
            rf = jnp.exp(jnp.minimum(maxm[J] - MI, 80.0))
            c = s * rf
            if J == I:
                c = jnp.where(causal, c, 0.0)
            ssum = ssum + jnp.sum(c, axis=1, keepdims=True)
            acc = acc + jnp.dot(c.astype(jnp.bfloat16), v_s[slj, :],
                                preferred_element_type=jnp.float32)
        norm = jnp.maximum(jnp.abs(ssum), nfI) + EPS
        hI = acc / norm
        mu = jnp.mean(hI, axis=1, keepdims=True)
        var = jnp.mean((hI - mu) * (hI - mu), axis=1, keepdims=True)
        hn = (hI - mu) * jax.lax.rsqrt(var + LN_EPS)
        o_ref[0, sli, :] = hn * gam_ref[0]


@jax.jit
def kernel(x, wq, wk, wv, wi, bi, wf, bf, ln_w):
    f32 = jnp.float32
    x = x.astype(f32)
    nhb = DH // BLK

    def _blockdiag(w):
        wt = w.astype(f32).reshape(NH, nhb, BLK, BLK).transpose(0, 1, 3, 2)
        p1 = jnp.pad(wt, ((0, 0), (0, 0), (0, 0), (0, (nhb - 1) * BLK)))
        p2 = p1.reshape(NH, nhb, BLK * DH)
        p3 = jnp.pad(p2, ((0, 0), (0, 0), (0, BLK)))
        p4 = p3.reshape(NH, nhb * (BLK * DH + BLK))[:, :DH * DH]
        return p4.reshape(NH, DH, DH)

    bf16 = jnp.bfloat16
    wq_d = _blockdiag(wq).astype(bf16)
    wk_d = _blockdiag(wk).astype(bf16)
    wv_d = _blockdiag(wv).astype(bf16)
    wi_eff = wi[:, :D] + wi[:, D:2 * D] + wi[:, 2 * D:]
    wf_eff = wf[:, :D] + wf[:, D:2 * D] + wf[:, 2 * D:]
    wg = jnp.concatenate([wi_eff, wf_eff], axis=0).T.astype(f32)
    bg = jnp.concatenate([bi, bf]).reshape(2 * NH, 1).astype(f32)
    gam = (1.0 + ln_w).reshape(NH, 1, DH).astype(f32)

    gstats = pl.pallas_call(
        _gate_kernel,
        out_shape=jax.ShapeDtypeStruct((B, NH, S, 3), f32),
        grid=(B,),
        in_specs=[
            pl.BlockSpec((1, S, D), lambda i: (i, 0, 0)),
            pl.BlockSpec((D, 2 * NH), lambda i: (0, 0)),
            pl.BlockSpec((2 * NH, 1), lambda i: (0, 0)),
        ],
        out_specs=pl.BlockSpec((1, NH, S, 3), lambda i: (i, 0, 0, 0)),
        compiler_params=pltpu.CompilerParams(
            dimension_semantics=("parallel",),
            vmem_limit_bytes=56 * 1024 * 1024,
        ),
        name="mlstm_gates",
    )(x, wg, bg)

    out = pl.pallas_call(
        _mlstm_kernel,
        out_shape=jax.ShapeDtypeStruct((B, S, D), f32),
        grid=(B * NH,),
        in_specs=[
            pl.BlockSpec((1, S, DH), lambda i: (i // NH, 0, i % NH)),
            pl.BlockSpec((1, DH, DH), lambda i: (i % NH, 0, 0)),
            pl.BlockSpec((1, DH, DH), lambda i: (i % NH, 0, 0)),
            pl.BlockSpec((1, DH, DH), lambda i: (i % NH, 0, 0)),
            pl.BlockSpec((1, 1, S, 3), lambda i: (i // NH, i % NH, 0, 0)),
            pl.BlockSpec((1, 1, DH), lambda i: (i % NH, 0, 0)),
        ],
        out_specs=pl.BlockSpec((1, S, DH), lambda i: (i // NH, 0, i % NH)),
        scratch_shapes=[
            pltpu.VMEM((S, DH), jnp.bfloat16),
            pltpu.VMEM((S, DH), jnp.bfloat16),
        ],
        compiler_params=pltpu.CompilerParams(
            dimension_semantics=("parallel",),
            vmem_limit_bytes=56 * 1024 * 1024,
        ),
        name="mlstm_attn",
    )(x, wq_d, wk_d, wv_d, gstats, gam)
    return out
